# baseline (speedup 1.0000x reference)
import functools

import jax
import jax.numpy as jnp
from jax import lax
from jax.experimental import pallas as pl
from jax.experimental.pallas import tpu as pltpu
from jax.experimental.pallas import tpu_sc as plsc

N = 32768
TARGET = int(0.99 * N)
W = N - TARGET + 1
CAP = 512
CAP_PAD = CAP + 16
HI_OFF = CAP - W
NROWS = 64
NC, NS = 2, 16
ROWS_PER_WORKER = NROWS // (NC * NS)
NV = N // 16

U = jnp.uint32


def _key_body(x_hbm, out_hbm, row_v, keys_v, h0, hl1, hh1, hl2, hh2, hl3, hh3,
              buf_lo, buf_hi, out_v):
    cid = lax.axis_index("c")
    sid = lax.axis_index("s")
    wid = sid * NC + cid
    ones = jnp.ones((16,), jnp.int32)
    zeros16 = jnp.zeros((16,), jnp.int32)
    iota = lax.iota(jnp.int32, 16)
    neg_inf = jnp.float32(-jnp.inf)
    pos_inf = jnp.float32(jnp.inf)

    def inv_to_float(k):
        top = k >> U(31)
        m2 = jnp.where(top == U(1), U(0x80000000), U(0xFFFFFFFF))
        return plsc.bitcast(k ^ m2, jnp.float32)

    def scan_hist(h, rank):
        def sbody(v, carry):
            found, base, tot = carry
            hv = h[pl.ds(v * 16, 16)]
            cin = plsc.cumsum(hv) + tot
            m = cin > rank
            has = jnp.max(m.astype(jnp.int32)) > 0
            lane = jnp.minimum(jnp.max(plsc.all_reduce_ffs(m)), 15)
            cumexcl = cin - hv
            baseval = jnp.max(jnp.where(iota == lane, cumexcl, jnp.int32(-(2**31))))
            upd = jnp.logical_and(found < 0, has)
            found = jnp.where(upd, v * 16 + lane, found)
            base = jnp.where(upd, baseval, base)
            return found, base, jnp.max(cin)

        f, b, _ = lax.fori_loop(
            0, 16, sbody, (jnp.int32(-1), jnp.int32(0), jnp.int32(0)))
        return f, b

    def bitonic_sort(buf):
        def ld(v):
            return buf[pl.ds(v * 16, 16)]

        def st(v, val):
            buf[pl.ds(v * 16, 16)] = val

        def vsort(v, descending):
            k = ld(v)
            ks, _ = plsc.sort_key_val(k, k, descending=descending)
            st(v, ks)

        for v in range(32):
            vsort(v, v % 2 == 1)
        for bk in (2, 4, 8, 16, 32):
            d = bk // 2
            while d >= 1:
                for base in range(0, 32, bk):
                    asc = (base // bk) % 2 == 0
                    for i0 in range(base, base + bk):
                        if (i0 - base) % (2 * d) < d:
                            va, vb = ld(i0), ld(i0 + d)
                            lo = jnp.minimum(va, vb)
                            hi = jnp.maximum(va, vb)
                            if asc:
                                st(i0, lo)
                                st(i0 + d, hi)
                            else:
                                st(i0, hi)
                                st(i0 + d, lo)
                d //= 2
            for v in range(32):
                vsort(v, descending=((v // bk) % 2 == 1) and bk < 32)

    def row_body(r, _):
        row = wid * ROWS_PER_WORKER + r

        def clr(i, c):
            for h in (h0, hl1, hh1, hl2, hh2, hl3, hh3):
                h[pl.ds(i * 16, 16)] = zeros16
            return c

        lax.fori_loop(0, 16, clr, 0)
        pltpu.sync_copy(x_hbm.at[row], row_v)

        def p1(i, c):
            f = row_v[pl.ds(i * 16, 16)]
            b = plsc.bitcast(f, U)
            negm = (b >> U(31)) * U(0xFFFFFFFF)
            key = b ^ (negm | U(0x80000000))
            keys_v[pl.ds(i * 16, 16)] = key
            byte = (key >> U(24)).astype(jnp.int32)
            plsc.addupdate_scatter(h0, [byte], ones)
            return c

        lax.fori_loop(0, NV, p1, 0, unroll=4)

        b_lo, base_lo = scan_hist(h0, jnp.int32(W - 1))
        b_hi, base_hi = scan_hist(h0, jnp.int32(TARGET - 1))
        pref_lo = b_lo.astype(U)
        pref_hi = b_hi.astype(U)
        r_lo = jnp.int32(W - 1) - base_lo
        r_hi = jnp.int32(TARGET - 1) - base_hi

        for lvl, (hl, hh) in enumerate(
                ((hl1, hh1), (hl2, hh2), (hl3, hh3)), start=1):
            shift = 24 - 8 * lvl

            def ph(i, c, shift=shift, hl=hl, hh=hh, pref_lo=pref_lo,
                   pref_hi=pref_hi):
                key = keys_v[pl.ds(i * 16, 16)]
                pk = key >> U(shift + 8)
                byte = ((key >> U(shift)) & U(0xFF)).astype(jnp.int32)
                plsc.addupdate_scatter(hl, [byte], ones, mask=pk == pref_lo)
                plsc.addupdate_scatter(hh, [byte], ones, mask=pk == pref_hi)
                return c

            lax.fori_loop(0, NV, ph, 0, unroll=4)
            b_lo, base_lo = scan_hist(hl, r_lo)
            b_hi, base_hi = scan_hist(hh, r_hi)
            pref_lo = (pref_lo << U(8)) | b_lo.astype(U)
            pref_hi = (pref_hi << U(8)) | b_hi.astype(U)
            r_lo = r_lo - base_lo
            r_hi = r_hi - base_hi

        k_lo = pref_lo
        k_hi = pref_hi

        k_lo_v = jnp.full((16,), k_lo, U)
        k_hi_v = jnp.full((16,), k_hi, U)

        def fill(i, c):
            buf_lo[pl.ds(i * 16, 16)] = k_lo_v
            buf_hi[pl.ds(i * 16, 16)] = k_hi_v
            return c

        lax.fori_loop(0, CAP_PAD // 16, fill, 0)

        def pg(i, carry):
            off_lo, off_hi = carry
            key = keys_v[pl.ds(i * 16, 16)]
            m_lo = key < k_lo
            m_hi = key > k_hi
            plsc.store_compressed(buf_lo.at[pl.ds(off_lo, 16)], key, mask=m_lo)
            plsc.store_compressed(buf_hi.at[pl.ds(off_hi, 16)], key, mask=m_hi)
            return (off_lo + jnp.sum(m_lo.astype(jnp.int32)),
                    off_hi + jnp.sum(m_hi.astype(jnp.int32)))

        lax.fori_loop(0, NV, pg, (jnp.int32(0), jnp.int32(0)), unroll=4)

        bitonic_sort(buf_lo)
        bitonic_sort(buf_hi)

        def am(i, carry):
            best, bl, br = carry
            lf = inv_to_float(buf_lo[pl.ds(i * 16, 16)])
            rf = inv_to_float(buf_hi[pl.ds(HI_OFF + i * 16, 16)])
            ln = rf - lf
            ln = jnp.where(i * 16 + iota < W, ln, pos_inf)
            vmin = jnp.min(ln)
            lane = jnp.minimum(jnp.max(plsc.all_reduce_ffs(ln == vmin)), 15)
            lval = jnp.max(jnp.where(iota == lane, lf, neg_inf))
            rval = jnp.max(jnp.where(iota == lane, rf, neg_inf))
            upd = vmin < best
            return (jnp.where(upd, vmin, best), jnp.where(upd, lval, bl),
                    jnp.where(upd, rval, br))

        _, best_l, best_r = lax.fori_loop(
            0, (W + 15) // 16, am, (pos_inf, jnp.float32(0), jnp.float32(0)))

        out_v[...] = jnp.where(iota == 0, best_l,
                               jnp.where(iota == 1, best_r, jnp.float32(0)))
        pltpu.sync_copy(out_v, out_hbm.at[row])
        return _

    lax.fori_loop(0, ROWS_PER_WORKER, row_body, 0)


@jax.jit
def kernel(x):
    mesh = plsc.VectorSubcoreMesh(core_axis_name="c", subcore_axis_name="s")
    run = pl.kernel(
        _key_body,
        out_type=jax.ShapeDtypeStruct((NROWS, 16), jnp.float32),
        mesh=mesh,
        compiler_params=pltpu.CompilerParams(needs_layout_passes=False),
        scratch_types=[
            pltpu.VMEM((N,), jnp.float32),
            pltpu.VMEM((N,), U),
            pltpu.VMEM((256,), jnp.int32),
            pltpu.VMEM((256,), jnp.int32),
            pltpu.VMEM((256,), jnp.int32),
            pltpu.VMEM((256,), jnp.int32),
            pltpu.VMEM((256,), jnp.int32),
            pltpu.VMEM((256,), jnp.int32),
            pltpu.VMEM((256,), jnp.int32),
            pltpu.VMEM((CAP_PAD,), U),
            pltpu.VMEM((CAP_PAD,), U),
            pltpu.VMEM((16,), jnp.float32),
        ],
    )
    out = run(x)
    return (out[:, 0], out[:, 1])


# speedup vs baseline: 2.1035x; 2.1035x over previous
import functools

import jax
import jax.numpy as jnp
from jax import lax
from jax.experimental import pallas as pl
from jax.experimental.pallas import tpu as pltpu
from jax.experimental.pallas import tpu_sc as plsc

N = 32768
TARGET = int(0.99 * N)
W = N - TARGET + 1
CAP = 512
CAP_PAD = CAP + 16
HI_OFF = CAP - W
NROWS = 64
NC, NS = 2, 16
ROWS_PER_WORKER = NROWS // (NC * NS)
NV = N // 16

U = jnp.uint32


def _key_body(x_hbm, out_hbm, row_v, keys_v, h0, hl1, hh1, hl2, hh2, hl3, hh3,
              buf_lo, buf_hi, out_v):
    cid = lax.axis_index("c")
    sid = lax.axis_index("s")
    wid = sid * NC + cid
    ones = jnp.ones((16,), jnp.int32)
    zeros16 = jnp.zeros((16,), jnp.int32)
    iota = lax.iota(jnp.int32, 16)
    neg_inf = jnp.float32(-jnp.inf)
    pos_inf = jnp.float32(jnp.inf)

    def inv_to_float(k):
        top = k >> U(31)
        m2 = jnp.where(top == U(1), U(0x80000000), U(0xFFFFFFFF))
        return plsc.bitcast(k ^ m2, jnp.float32)

    def scan_hist(h, rank):
        def sbody(v, carry):
            found, base, tot = carry
            hv = h[pl.ds(v * 16, 16)]
            cin = plsc.cumsum(hv) + tot
            m = cin > rank
            has = jnp.max(m.astype(jnp.int32)) > 0
            lane = jnp.minimum(jnp.max(plsc.all_reduce_ffs(m)), 15)
            cumexcl = cin - hv
            baseval = jnp.max(jnp.where(iota == lane, cumexcl, jnp.int32(-(2**31))))
            upd = jnp.logical_and(found < 0, has)
            found = jnp.where(upd, v * 16 + lane, found)
            base = jnp.where(upd, baseval, base)
            return found, base, jnp.max(cin)

        f, b, _ = lax.fori_loop(
            0, 16, sbody, (jnp.int32(-1), jnp.int32(0), jnp.int32(0)))
        return f, b

    def bitonic_sort(buf):
        def ld(v):
            return buf[pl.ds(v * 16, 16)]

        def st(v, val):
            buf[pl.ds(v * 16, 16)] = val

        def vsort(v, descending):
            k = ld(v)
            ks, _ = plsc.sort_key_val(k, k, descending=descending)
            st(v, ks)

        for v in range(32):
            vsort(v, v % 2 == 1)
        for bk in (2, 4, 8, 16, 32):
            d = bk // 2
            while d >= 1:
                for base in range(0, 32, bk):
                    asc = (base // bk) % 2 == 0
                    for i0 in range(base, base + bk):
                        if (i0 - base) % (2 * d) < d:
                            va, vb = ld(i0), ld(i0 + d)
                            lo = jnp.minimum(va, vb)
                            hi = jnp.maximum(va, vb)
                            if asc:
                                st(i0, lo)
                                st(i0 + d, hi)
                            else:
                                st(i0, hi)
                                st(i0 + d, lo)
                d //= 2
            for v in range(32):
                vsort(v, descending=((v // bk) % 2 == 1) and bk < 32)

    def row_body(r, _):
        row = wid * ROWS_PER_WORKER + r

        def clr(i, c):
            for h in (h0, hl1, hh1, hl2, hh2, hl3, hh3):
                h[pl.ds(i * 16, 16)] = zeros16
            return c

        lax.fori_loop(0, 16, clr, 0)
        pltpu.sync_copy(x_hbm.at[row], row_v)

        def p1(i):
            f = row_v[pl.ds(i * 16, 16)]
            b = plsc.bitcast(f, U)
            negm = (b >> U(31)) * U(0xFFFFFFFF)
            key = b ^ (negm | U(0x80000000))
            keys_v[pl.ds(i * 16, 16)] = key
            byte = (key >> U(24)).astype(jnp.int32)
            plsc.addupdate_scatter(h0, [byte], ones)

        plsc.parallel_loop(0, NV, unroll=4)(p1)

        b_lo, base_lo = scan_hist(h0, jnp.int32(W - 1))
        b_hi, base_hi = scan_hist(h0, jnp.int32(TARGET - 1))
        pref_lo = b_lo.astype(U)
        pref_hi = b_hi.astype(U)
        r_lo = jnp.int32(W - 1) - base_lo
        r_hi = jnp.int32(TARGET - 1) - base_hi

        for lvl, (hl, hh) in enumerate(
                ((hl1, hh1), (hl2, hh2), (hl3, hh3)), start=1):
            shift = 24 - 8 * lvl

            def ph(i, shift=shift, hl=hl, hh=hh, pref_lo=pref_lo,
                   pref_hi=pref_hi):
                key = keys_v[pl.ds(i * 16, 16)]
                pk = key >> U(shift + 8)
                byte = ((key >> U(shift)) & U(0xFF)).astype(jnp.int32)
                plsc.addupdate_scatter(hl, [byte], ones, mask=pk == pref_lo)
                plsc.addupdate_scatter(hh, [byte], ones, mask=pk == pref_hi)

            plsc.parallel_loop(0, NV, unroll=4)(ph)
            b_lo, base_lo = scan_hist(hl, r_lo)
            b_hi, base_hi = scan_hist(hh, r_hi)
            pref_lo = (pref_lo << U(8)) | b_lo.astype(U)
            pref_hi = (pref_hi << U(8)) | b_hi.astype(U)
            r_lo = r_lo - base_lo
            r_hi = r_hi - base_hi

        k_lo = pref_lo
        k_hi = pref_hi

        k_lo_v = jnp.full((16,), k_lo, U)
        k_hi_v = jnp.full((16,), k_hi, U)

        def fill(i, c):
            buf_lo[pl.ds(i * 16, 16)] = k_lo_v
            buf_hi[pl.ds(i * 16, 16)] = k_hi_v
            return c

        lax.fori_loop(0, CAP_PAD // 16, fill, 0)

        def pg(i, carry):
            off_lo, off_hi = carry
            key = keys_v[pl.ds(i * 16, 16)]
            m_lo = key < k_lo
            m_hi = key > k_hi
            plsc.store_compressed(buf_lo.at[pl.ds(off_lo, 16)], key, mask=m_lo)
            plsc.store_compressed(buf_hi.at[pl.ds(off_hi, 16)], key, mask=m_hi)
            return (off_lo + jnp.sum(m_lo.astype(jnp.int32)),
                    off_hi + jnp.sum(m_hi.astype(jnp.int32)))

        lax.fori_loop(0, NV, pg, (jnp.int32(0), jnp.int32(0)))

        bitonic_sort(buf_lo)
        bitonic_sort(buf_hi)

        def am(i, carry):
            best, bl, br = carry
            lf = inv_to_float(buf_lo[pl.ds(i * 16, 16)])
            rf = inv_to_float(buf_hi[pl.ds(HI_OFF + i * 16, 16)])
            ln = rf - lf
            ln = jnp.where(i * 16 + iota < W, ln, pos_inf)
            vmin = jnp.min(ln)
            lane = jnp.minimum(jnp.max(plsc.all_reduce_ffs(ln == vmin)), 15)
            lval = jnp.max(jnp.where(iota == lane, lf, neg_inf))
            rval = jnp.max(jnp.where(iota == lane, rf, neg_inf))
            upd = vmin < best
            return (jnp.where(upd, vmin, best), jnp.where(upd, lval, bl),
                    jnp.where(upd, rval, br))

        _, best_l, best_r = lax.fori_loop(
            0, (W + 15) // 16, am, (pos_inf, jnp.float32(0), jnp.float32(0)))

        out_v[...] = jnp.where(iota == 0, best_l,
                               jnp.where(iota == 1, best_r, jnp.float32(0)))
        pltpu.sync_copy(out_v, out_hbm.at[row])
        return _

    lax.fori_loop(0, ROWS_PER_WORKER, row_body, 0)


@jax.jit
def kernel(x):
    mesh = plsc.VectorSubcoreMesh(core_axis_name="c", subcore_axis_name="s")
    run = pl.kernel(
        _key_body,
        out_type=jax.ShapeDtypeStruct((NROWS, 16), jnp.float32),
        mesh=mesh,
        compiler_params=pltpu.CompilerParams(needs_layout_passes=False),
        scratch_types=[
            pltpu.VMEM((N,), jnp.float32),
            pltpu.VMEM((N,), U),
            pltpu.VMEM((256,), jnp.int32),
            pltpu.VMEM((256,), jnp.int32),
            pltpu.VMEM((256,), jnp.int32),
            pltpu.VMEM((256,), jnp.int32),
            pltpu.VMEM((256,), jnp.int32),
            pltpu.VMEM((256,), jnp.int32),
            pltpu.VMEM((256,), jnp.int32),
            pltpu.VMEM((CAP_PAD,), U),
            pltpu.VMEM((CAP_PAD,), U),
            pltpu.VMEM((16,), jnp.float32),
        ],
    )
    out = run(x)
    return (out[:, 0], out[:, 1])


# speedup vs baseline: 2.6081x; 1.2399x over previous
import functools

import jax
import jax.numpy as jnp
from jax import lax
from jax.experimental import pallas as pl
from jax.experimental.pallas import tpu as pltpu
from jax.experimental.pallas import tpu_sc as plsc

N = 32768
TARGET = int(0.99 * N)
W = N - TARGET + 1
CAP = 512
CAP_PAD = CAP + 16
HI_OFF = CAP - W
NROWS = 64
NC, NS = 2, 16
ROWS_PER_WORKER = NROWS // (NC * NS)
NV = N // 16

U = jnp.uint32


def _key_body(x_hbm, out_hbm, row_v, keys_v, h0, hl1, hh1, hl2, hh2, hl3, hh3,
              buf_lo, buf_hi, out_v):
    cid = lax.axis_index("c")
    sid = lax.axis_index("s")
    wid = sid * NC + cid
    ones = jnp.ones((16,), jnp.int32)
    zeros16 = jnp.zeros((16,), jnp.int32)
    iota = lax.iota(jnp.int32, 16)
    neg_inf = jnp.float32(-jnp.inf)
    pos_inf = jnp.float32(jnp.inf)

    def inv_to_float(k):
        top = k >> U(31)
        m2 = jnp.where(top == U(1), U(0x80000000), U(0xFFFFFFFF))
        return plsc.bitcast(k ^ m2, jnp.float32)

    def scan_hist(h, rank):
        def sbody(v, carry):
            found, base, tot = carry
            v = v.astype(jnp.int32)
            hv = h[pl.ds(v * 16, 16)]
            cin = plsc.cumsum(hv) + tot
            m = cin > rank
            has = jnp.max(m.astype(jnp.int32)) > 0
            lane = jnp.minimum(jnp.max(plsc.all_reduce_ffs(m)), 15)
            cumexcl = cin - hv
            baseval = jnp.max(jnp.where(iota == lane, cumexcl, jnp.int32(-(2**31))))
            upd = jnp.logical_and(found < 0, has)
            found = jnp.where(upd, v * 16 + lane, found)
            base = jnp.where(upd, baseval, base)
            return found, base, jnp.max(cin)

        f, b, _ = plsc.parallel_loop(
            0, 16, unroll=4,
            carry=(jnp.int32(-1), jnp.int32(0), jnp.int32(0)))(sbody)
        return f, b

    def bitonic_sort(buf):
        def ld(v):
            return buf[pl.ds(v * 16, 16)]

        def st(v, val):
            buf[pl.ds(v * 16, 16)] = val

        def vsort(v, descending):
            k = ld(v)
            ks, _ = plsc.sort_key_val(k, k, descending=descending)
            st(v, ks)

        for v in range(32):
            vsort(v, v % 2 == 1)
        for bk in (2, 4, 8, 16, 32):
            d = bk // 2
            while d >= 1:
                for base in range(0, 32, bk):
                    asc = (base // bk) % 2 == 0
                    for i0 in range(base, base + bk):
                        if (i0 - base) % (2 * d) < d:
                            va, vb = ld(i0), ld(i0 + d)
                            lo = jnp.minimum(va, vb)
                            hi = jnp.maximum(va, vb)
                            if asc:
                                st(i0, lo)
                                st(i0 + d, hi)
                            else:
                                st(i0, hi)
                                st(i0 + d, lo)
                d //= 2
            for v in range(32):
                vsort(v, descending=((v // bk) % 2 == 1) and bk < 32)

    def row_body(r, _):
        row = wid * ROWS_PER_WORKER + r

        def clr(i):
            for h in (h0, hl1, hh1, hl2, hh2, hl3, hh3):
                h[pl.ds(i * 16, 16)] = zeros16

        plsc.parallel_loop(0, 16, unroll=2)(clr)
        pltpu.sync_copy(x_hbm.at[row], row_v)

        def p1(i):
            f = row_v[pl.ds(i * 16, 16)]
            b = plsc.bitcast(f, U)
            negm = (b >> U(31)) * U(0xFFFFFFFF)
            key = b ^ (negm | U(0x80000000))
            keys_v[pl.ds(i * 16, 16)] = key
            byte = (key >> U(24)).astype(jnp.int32)
            plsc.addupdate_scatter(h0, [byte], ones)

        plsc.parallel_loop(0, NV, unroll=4)(p1)

        b_lo, base_lo = scan_hist(h0, jnp.int32(W - 1))
        b_hi, base_hi = scan_hist(h0, jnp.int32(TARGET - 1))
        pref_lo = b_lo.astype(U)
        pref_hi = b_hi.astype(U)
        r_lo = jnp.int32(W - 1) - base_lo
        r_hi = jnp.int32(TARGET - 1) - base_hi

        for lvl, (hl, hh) in enumerate(
                ((hl1, hh1), (hl2, hh2), (hl3, hh3)), start=1):
            shift = 24 - 8 * lvl

            def ph(i, shift=shift, hl=hl, hh=hh, pref_lo=pref_lo,
                   pref_hi=pref_hi):
                key = keys_v[pl.ds(i * 16, 16)]
                pk = key >> U(shift + 8)
                byte = ((key >> U(shift)) & U(0xFF)).astype(jnp.int32)
                plsc.addupdate_scatter(hl, [byte], ones, mask=pk == pref_lo)
                plsc.addupdate_scatter(hh, [byte], ones, mask=pk == pref_hi)

            plsc.parallel_loop(0, NV, unroll=4)(ph)
            b_lo, base_lo = scan_hist(hl, r_lo)
            b_hi, base_hi = scan_hist(hh, r_hi)
            pref_lo = (pref_lo << U(8)) | b_lo.astype(U)
            pref_hi = (pref_hi << U(8)) | b_hi.astype(U)
            r_lo = r_lo - base_lo
            r_hi = r_hi - base_hi

        k_lo = pref_lo
        k_hi = pref_hi

        k_lo_v = jnp.full((16,), k_lo, U)
        k_hi_v = jnp.full((16,), k_hi, U)

        def fill(i):
            buf_lo[pl.ds(i * 16, 16)] = k_lo_v
            buf_hi[pl.ds(i * 16, 16)] = k_hi_v

        plsc.parallel_loop(0, CAP_PAD // 16, unroll=3)(fill)

        def pg(i, carry):
            off_lo, off_hi = carry
            key = keys_v[pl.ds(i * 16, 16)]
            m_lo = key < k_lo
            m_hi = key > k_hi
            plsc.store_compressed(buf_lo.at[pl.ds(off_lo, 16)], key, mask=m_lo)
            plsc.store_compressed(buf_hi.at[pl.ds(off_hi, 16)], key, mask=m_hi)
            return (off_lo + jnp.sum(m_lo.astype(jnp.int32)),
                    off_hi + jnp.sum(m_hi.astype(jnp.int32)))

        plsc.parallel_loop(
            0, NV, unroll=4, carry=(jnp.int32(0), jnp.int32(0)))(pg)

        bitonic_sort(buf_lo)
        bitonic_sort(buf_hi)

        def am(i, carry):
            best, bl, br = carry
            i = i.astype(jnp.int32)
            lf = inv_to_float(buf_lo[pl.ds(i * 16, 16)])
            rf = inv_to_float(buf_hi[pl.ds(HI_OFF + i * 16, 16)])
            ln = rf - lf
            ln = jnp.where(i * 16 + iota < W, ln, pos_inf)
            vmin = jnp.min(ln)
            lane = jnp.minimum(jnp.max(plsc.all_reduce_ffs(ln == vmin)), 15)
            lval = jnp.max(jnp.where(iota == lane, lf, neg_inf))
            rval = jnp.max(jnp.where(iota == lane, rf, neg_inf))
            upd = vmin < best
            return (jnp.where(upd, vmin, best), jnp.where(upd, lval, bl),
                    jnp.where(upd, rval, br))

        _, best_l, best_r = plsc.parallel_loop(
            0, (W + 15) // 16, unroll=3,
            carry=(pos_inf, jnp.float32(0), jnp.float32(0)))(am)

        out_v[...] = jnp.where(iota == 0, best_l,
                               jnp.where(iota == 1, best_r, jnp.float32(0)))
        pltpu.sync_copy(out_v, out_hbm.at[row])
        return _

    lax.fori_loop(0, ROWS_PER_WORKER, row_body, 0)


@jax.jit
def kernel(x):
    mesh = plsc.VectorSubcoreMesh(core_axis_name="c", subcore_axis_name="s")
    run = pl.kernel(
        _key_body,
        out_type=jax.ShapeDtypeStruct((NROWS, 16), jnp.float32),
        mesh=mesh,
        compiler_params=pltpu.CompilerParams(needs_layout_passes=False),
        scratch_types=[
            pltpu.VMEM((N,), jnp.float32),
            pltpu.VMEM((N,), U),
            pltpu.VMEM((256,), jnp.int32),
            pltpu.VMEM((256,), jnp.int32),
            pltpu.VMEM((256,), jnp.int32),
            pltpu.VMEM((256,), jnp.int32),
            pltpu.VMEM((256,), jnp.int32),
            pltpu.VMEM((256,), jnp.int32),
            pltpu.VMEM((256,), jnp.int32),
            pltpu.VMEM((CAP_PAD,), U),
            pltpu.VMEM((CAP_PAD,), U),
            pltpu.VMEM((16,), jnp.float32),
        ],
    )
    out = run(x)
    return (out[:, 0], out[:, 1])
